# baseline (speedup 1.0000x reference)
import jax
import jax.numpy as jnp
from jax import lax
from jax.experimental import pallas as pl
from jax.experimental.pallas import tpu as pltpu
from jax.experimental.pallas import tpu_sc as plsc

N_VOCAB_ = 49408
N_EMBD_ = 768
N_TOKEN_ = 77
BATCH_ = 1024

NC = 2
NS = 16
LANES = 16
NW = NC * NS

B_PER_W = BATCH_ // NW
NB_CH = 4
TB = 8
NBAND = 10
CH_PER_BAND = B_PER_W // NB_CH
NCH = NBAND * CH_PER_BAND
ROWS = NB_CH * TB
CW = 5
D_SLICES = N_EMBD_ // LANES
NSLOT = (NCH + 2) // 3 * 3


def _body(idx_hbm, tab_hbm, pos_hbm, out_hbm,
          idx_v, pos_v, buf0, buf1, buf2, bufW,
          sg0, sg1, sg2, sw0, sw1, sw2, swW):
    wid = lax.axis_index("s") * NC + lax.axis_index("c")
    base_batch = wid * B_PER_W

    bufs = (buf0, buf1, buf2)
    sgs = (sg0, sg1, sg2)
    sws = (sw0, sw1, sw2)

    pltpu.sync_copy(idx_hbm.at[wid], idx_v)

    def start_gather(q, buf, sem):
        pltpu.async_copy(tab_hbm.at[idx_v.at[q]], buf, sem)

    def step(q, s):
        buf = bufs[s]
        k = q // CH_PER_BAND
        c = lax.rem(q, CH_PER_BAND)
        bb = base_batch + c * NB_CH
        t0 = pl.multiple_of(k * TB, TB)

        @pl.when(q < NCH)
        def _():
            @pl.when(c == 0)
            def _():
                pltpu.sync_copy(pos_hbm.at[pl.ds(t0, TB)], pos_v)

            pltpu.make_async_copy(tab_hbm.at[idx_v.at[q]], buf, sgs[s]).wait()

        @pl.when(q + 2 < NCH)
        def _():
            s2 = (s + 2) % 3

            @pl.when(jnp.logical_and(q >= 1, q <= NCH - TB))
            def _():
                for i in range(NB_CH):
                    pltpu.make_async_copy(
                        bufs[s2].at[pl.ds(i * TB, TB)],
                        out_hbm.at[base_batch, pl.ds(0, TB)],
                        sws[s2]).wait()

            start_gather(q + 2, bufs[s2], sgs[s2])

        @pl.when(q < NCH)
        def _():
            @pl.when(k < NBAND - 1)
            def _():
                for i in range(NB_CH):
                    def add_body(j, _, i=i):
                        for u in range(2):
                            sl = pl.ds((2 * j + u) * LANES, LANES)
                            for r8 in range(TB):
                                plsc.addupdate(buf.at[i * TB + r8, sl],
                                               pos_v[r8, sl])
                        return 0

                    lax.fori_loop(0, D_SLICES // 2, add_body, 0)
                    pltpu.async_copy(buf.at[pl.ds(i * TB, TB)],
                                     out_hbm.at[bb + i, pl.ds(t0, TB)],
                                     sws[s])

            @pl.when(k == NBAND - 1)
            def _():
                @pl.when(q > NCH - CH_PER_BAND)
                def _():
                    for i in range(NB_CH):
                        pltpu.make_async_copy(
                            bufW.at[i],
                            out_hbm.at[base_batch, pl.ds(N_TOKEN_ - CW, CW)],
                            swW).wait()

                for i in range(NB_CH):
                    def tail_body(j, _, i=i):
                        for u in range(2):
                            sl = pl.ds((2 * j + u) * LANES, LANES)
                            for r in range(CW):
                                bufW[i, r, sl] = (buf[i * TB + r, sl]
                                                  + pos_v[r, sl])
                        return 0

                    lax.fori_loop(0, D_SLICES // 2, tail_body, 0)
                    pltpu.async_copy(bufW.at[i],
                                     out_hbm.at[bb + i,
                                                pl.ds(N_TOKEN_ - CW, CW)],
                                     swW)

    def triple_body(p, _):
        for s in range(3):
            step(p * 3 + s, s)
        return 0

    start_gather(0, buf0, sg0)
    start_gather(1, buf1, sg1)
    lax.fori_loop(0, NSLOT // 3, triple_body, 0)
    for i in range(NB_CH):
        pltpu.make_async_copy(bufW.at[i],
                              out_hbm.at[base_batch, pl.ds(N_TOKEN_ - CW, CW)],
                              swW).wait()


@jax.jit
def kernel(tokens, embedding_token, embedding_posicao):
    mesh = plsc.VectorSubcoreMesh(core_axis_name="c", subcore_axis_name="s")
    tok = tokens.astype(jnp.int32)
    tok_pad = jnp.concatenate([tok, tok[:, N_TOKEN_ - 3:]], axis=1)
    idx = tok_pad.reshape(NW, CH_PER_BAND, NB_CH, NBAND, TB)
    idx = idx.transpose(0, 3, 1, 2, 4).reshape(NW, NCH, ROWS)
    pos_pad = jnp.concatenate(
        [embedding_posicao,
         jnp.zeros((NBAND * TB - N_TOKEN_, N_EMBD_), jnp.float32)], axis=0)
    out = pl.kernel(
        _body,
        out_type=jax.ShapeDtypeStruct((BATCH_, N_TOKEN_, N_EMBD_), jnp.float32),
        mesh=mesh,
        scratch_types=[
            pltpu.VMEM((NCH, ROWS), jnp.int32),
            pltpu.VMEM((TB, N_EMBD_), jnp.float32),
            pltpu.VMEM((ROWS, N_EMBD_), jnp.float32),
            pltpu.VMEM((ROWS, N_EMBD_), jnp.float32),
            pltpu.VMEM((ROWS, N_EMBD_), jnp.float32),
            pltpu.VMEM((NB_CH, CW, N_EMBD_), jnp.float32),
            pltpu.SemaphoreType.DMA,
            pltpu.SemaphoreType.DMA,
            pltpu.SemaphoreType.DMA,
            pltpu.SemaphoreType.DMA,
            pltpu.SemaphoreType.DMA,
            pltpu.SemaphoreType.DMA,
            pltpu.SemaphoreType.DMA,
        ],
    )(idx, embedding_token, pos_pad)
    return out


# speedup vs baseline: 1.6714x; 1.6714x over previous
import jax
import jax.numpy as jnp
from jax import lax
from jax.experimental import pallas as pl
from jax.experimental.pallas import tpu as pltpu
from jax.experimental.pallas import tpu_sc as plsc

N_VOCAB_ = 49408
N_EMBD_ = 768
N_TOKEN_ = 77
BATCH_ = 1024

NC = 2
NS = 16
LANES = 16
NW = NC * NS

B_PER_W = BATCH_ // NW
NB_CH = 4
TB = 8
NBAND = 10
CH_PER_BAND = B_PER_W // NB_CH
NCH = NBAND * CH_PER_BAND
ROWS = NB_CH * TB
CW = 5
D_SLICES = N_EMBD_ // LANES
NSLOT = (NCH + 2) // 3 * 3


def _body(idx_hbm, tab_hbm, pos_hbm, out_hbm,
          idx_v, pos_v, buf0, buf1, buf2, bufW,
          sg0, sg1, sg2, sw0, sw1, sw2, swW):
    wid = lax.axis_index("s") * NC + lax.axis_index("c")
    base_batch = wid * B_PER_W

    bufs = (buf0, buf1, buf2)
    sgs = (sg0, sg1, sg2)
    sws = (sw0, sw1, sw2)

    pltpu.sync_copy(idx_hbm.at[wid], idx_v)

    def start_gather(q, buf, sem):
        pltpu.async_copy(tab_hbm.at[idx_v.at[q]], buf, sem)

    def step(q, s):
        buf = bufs[s]
        k = q // CH_PER_BAND
        c = lax.rem(q, CH_PER_BAND)
        bb = base_batch + c * NB_CH
        t0 = pl.multiple_of(k * TB, TB)

        @pl.when(q < NCH)
        def _():
            @pl.when(c == 0)
            def _():
                pltpu.sync_copy(pos_hbm.at[pl.ds(t0, TB)], pos_v)

            pltpu.make_async_copy(tab_hbm.at[idx_v.at[q]], buf, sgs[s]).wait()

            @pl.when(k < NBAND - 1)
            def _():
                def add_body(j, _):
                    for u in range(2):
                        sl = pl.ds((2 * j + u) * LANES, LANES)
                        for r8 in range(TB):
                            v = pos_v[r8, sl]
                            for i in range(NB_CH):
                                plsc.addupdate(buf.at[i * TB + r8, sl], v)
                    return 0

                lax.fori_loop(0, D_SLICES // 2, add_body, 0)
                for i in range(NB_CH):
                    pltpu.async_copy(buf.at[pl.ds(i * TB, TB)],
                                     out_hbm.at[bb + i, pl.ds(t0, TB)],
                                     sws[s])

            @pl.when(k == NBAND - 1)
            def _():
                @pl.when(q > NCH - CH_PER_BAND)
                def _():
                    for i in range(NB_CH):
                        pltpu.make_async_copy(
                            bufW.at[i],
                            out_hbm.at[base_batch, pl.ds(N_TOKEN_ - CW, CW)],
                            swW).wait()

                def tail_body(j, _):
                    sl = pl.ds(j * LANES, LANES)
                    for r in range(CW):
                        v = pos_v[r, sl]
                        for i in range(NB_CH):
                            bufW[i, r, sl] = buf[i * TB + r, sl] + v
                    return 0

                lax.fori_loop(0, D_SLICES, tail_body, 0)
                for i in range(NB_CH):
                    pltpu.async_copy(bufW.at[i],
                                     out_hbm.at[bb + i,
                                                pl.ds(N_TOKEN_ - CW, CW)],
                                     swW)

        @pl.when(q + 2 < NCH)
        def _():
            s2 = (s + 2) % 3

            @pl.when(jnp.logical_and(q >= 1, q <= NCH - TB))
            def _():
                for i in range(NB_CH):
                    pltpu.make_async_copy(
                        bufs[s2].at[pl.ds(i * TB, TB)],
                        out_hbm.at[base_batch, pl.ds(0, TB)],
                        sws[s2]).wait()

            start_gather(q + 2, bufs[s2], sgs[s2])

    def triple_body(p, _):
        for s in range(3):
            step(p * 3 + s, s)
        return 0

    start_gather(0, buf0, sg0)
    start_gather(1, buf1, sg1)
    lax.fori_loop(0, NSLOT // 3, triple_body, 0)
    for i in range(NB_CH):
        pltpu.make_async_copy(bufW.at[i],
                              out_hbm.at[base_batch, pl.ds(N_TOKEN_ - CW, CW)],
                              swW).wait()


@jax.jit
def kernel(tokens, embedding_token, embedding_posicao):
    mesh = plsc.VectorSubcoreMesh(core_axis_name="c", subcore_axis_name="s")
    tok = tokens.astype(jnp.int32)
    tok_pad = jnp.concatenate([tok, tok[:, N_TOKEN_ - 3:]], axis=1)
    idx = tok_pad.reshape(NW, CH_PER_BAND, NB_CH, NBAND, TB)
    idx = idx.transpose(0, 3, 1, 2, 4).reshape(NW, NCH, ROWS)
    pos_pad = jnp.concatenate(
        [embedding_posicao,
         jnp.zeros((NBAND * TB - N_TOKEN_, N_EMBD_), jnp.float32)], axis=0)
    out = pl.kernel(
        _body,
        out_type=jax.ShapeDtypeStruct((BATCH_, N_TOKEN_, N_EMBD_), jnp.float32),
        mesh=mesh,
        scratch_types=[
            pltpu.VMEM((NCH, ROWS), jnp.int32),
            pltpu.VMEM((TB, N_EMBD_), jnp.float32),
            pltpu.VMEM((ROWS, N_EMBD_), jnp.float32),
            pltpu.VMEM((ROWS, N_EMBD_), jnp.float32),
            pltpu.VMEM((ROWS, N_EMBD_), jnp.float32),
            pltpu.VMEM((NB_CH, CW, N_EMBD_), jnp.float32),
            pltpu.SemaphoreType.DMA,
            pltpu.SemaphoreType.DMA,
            pltpu.SemaphoreType.DMA,
            pltpu.SemaphoreType.DMA,
            pltpu.SemaphoreType.DMA,
            pltpu.SemaphoreType.DMA,
            pltpu.SemaphoreType.DMA,
        ],
    )(idx, embedding_token, pos_pad)
    return out


# speedup vs baseline: 1.7131x; 1.0249x over previous
import jax
import jax.numpy as jnp
from jax import lax
from jax.experimental import pallas as pl
from jax.experimental.pallas import tpu as pltpu
from jax.experimental.pallas import tpu_sc as plsc

N_VOCAB_ = 49408
N_EMBD_ = 768
N_TOKEN_ = 77
BATCH_ = 1024

NC = 2
NS = 16
LANES = 16
NW = NC * NS

B_PER_W = BATCH_ // NW
NB_CH = 4
TB = 8
NBAND = 10
CH_PER_BAND = B_PER_W // NB_CH
NCH = NBAND * CH_PER_BAND
ROWS = NB_CH * TB
CW = 5
D_SLICES = N_EMBD_ // LANES
NSLOT = (NCH + 2) // 3 * 3


def _body(idx_hbm, tab_hbm, pos_hbm, out_hbm,
          idx_v, pos_v, buf0, buf1, buf2, bufW,
          sg0, sg1, sg2, sw0, sw1, sw2, swW):
    wid = lax.axis_index("s") * NC + lax.axis_index("c")
    base_batch = wid * B_PER_W

    bufs = (buf0, buf1, buf2)
    sgs = (sg0, sg1, sg2)
    sws = (sw0, sw1, sw2)

    pltpu.sync_copy(idx_hbm.at[wid], idx_v)

    def start_gather(q, buf, sem):
        pltpu.async_copy(tab_hbm.at[idx_v.at[q]], buf, sem)

    def step(q, s):
        buf = bufs[s]
        k = q // CH_PER_BAND
        c = lax.rem(q, CH_PER_BAND)
        bb = base_batch + c * NB_CH
        t0 = pl.multiple_of(k * TB, TB)

        @pl.when(q < NCH)
        def _():
            @pl.when(c == 0)
            def _():
                pltpu.sync_copy(pos_hbm.at[pl.ds(t0, TB)], pos_v)

            pltpu.make_async_copy(tab_hbm.at[idx_v.at[q]], buf, sgs[s]).wait()

            @pl.when(k < NBAND - 1)
            def _():
                def add_body(j, _):
                    sl = pl.ds(j * LANES, LANES)
                    for r8 in range(TB):
                        v = pos_v[r8, sl]
                        for i in range(NB_CH):
                            plsc.addupdate(buf.at[i * TB + r8, sl], v)
                    return 0

                lax.fori_loop(0, D_SLICES, add_body, 0)
                for i in range(NB_CH):
                    pltpu.async_copy(buf.at[pl.ds(i * TB, TB)],
                                     out_hbm.at[bb + i, pl.ds(t0, TB)],
                                     sws[s])

            @pl.when(k == NBAND - 1)
            def _():
                @pl.when(q > NCH - CH_PER_BAND)
                def _():
                    for i in range(NB_CH):
                        pltpu.make_async_copy(
                            bufW.at[i],
                            out_hbm.at[base_batch, pl.ds(N_TOKEN_ - CW, CW)],
                            swW).wait()

                def tail_body(j, _):
                    sl = pl.ds(j * LANES, LANES)
                    for r in range(CW):
                        v = pos_v[r, sl]
                        for i in range(NB_CH):
                            bufW[i, r, sl] = buf[i * TB + r, sl] + v
                    return 0

                lax.fori_loop(0, D_SLICES, tail_body, 0)
                for i in range(NB_CH):
                    pltpu.async_copy(bufW.at[i],
                                     out_hbm.at[bb + i,
                                                pl.ds(N_TOKEN_ - CW, CW)],
                                     swW)

        @pl.when(q + 2 < NCH)
        def _():
            s2 = (s + 2) % 3

            @pl.when(jnp.logical_and(q >= 1, q <= NCH - TB))
            def _():
                for i in range(NB_CH):
                    pltpu.make_async_copy(
                        bufs[s2].at[pl.ds(i * TB, TB)],
                        out_hbm.at[base_batch, pl.ds(0, TB)],
                        sws[s2]).wait()

            start_gather(q + 2, bufs[s2], sgs[s2])

    def triple_body(p, _):
        for s in range(3):
            step(p * 3 + s, s)
        return 0

    start_gather(0, buf0, sg0)
    start_gather(1, buf1, sg1)
    lax.fori_loop(0, NSLOT // 3, triple_body, 0)
    for i in range(NB_CH):
        pltpu.make_async_copy(bufW.at[i],
                              out_hbm.at[base_batch, pl.ds(N_TOKEN_ - CW, CW)],
                              swW).wait()


@jax.jit
def kernel(tokens, embedding_token, embedding_posicao):
    mesh = plsc.VectorSubcoreMesh(core_axis_name="c", subcore_axis_name="s")
    tok = tokens.astype(jnp.int32)
    tok_pad = jnp.concatenate([tok, tok[:, N_TOKEN_ - 3:]], axis=1)
    idx = tok_pad.reshape(NW, CH_PER_BAND, NB_CH, NBAND, TB)
    idx = idx.transpose(0, 3, 1, 2, 4).reshape(NW, NCH, ROWS)
    pos_pad = jnp.concatenate(
        [embedding_posicao,
         jnp.zeros((NBAND * TB - N_TOKEN_, N_EMBD_), jnp.float32)], axis=0)
    out = pl.kernel(
        _body,
        out_type=jax.ShapeDtypeStruct((BATCH_, N_TOKEN_, N_EMBD_), jnp.float32),
        mesh=mesh,
        scratch_types=[
            pltpu.VMEM((NCH, ROWS), jnp.int32),
            pltpu.VMEM((TB, N_EMBD_), jnp.float32),
            pltpu.VMEM((ROWS, N_EMBD_), jnp.float32),
            pltpu.VMEM((ROWS, N_EMBD_), jnp.float32),
            pltpu.VMEM((ROWS, N_EMBD_), jnp.float32),
            pltpu.VMEM((NB_CH, CW, N_EMBD_), jnp.float32),
            pltpu.SemaphoreType.DMA,
            pltpu.SemaphoreType.DMA,
            pltpu.SemaphoreType.DMA,
            pltpu.SemaphoreType.DMA,
            pltpu.SemaphoreType.DMA,
            pltpu.SemaphoreType.DMA,
            pltpu.SemaphoreType.DMA,
        ],
    )(idx, embedding_token, pos_pad)
    return out
